# Initial kernel scaffold; baseline (speedup 1.0000x reference)
import jax
import jax.numpy as jnp
from jax.experimental import pallas as pl


def kernel(edge_index, user_emb_weight, item_emb_weight):
    raise NotImplementedError("write your pallas kernel here")


# speedup vs baseline: 7.2684x; 7.2684x over previous
import functools

import jax
import jax.numpy as jnp
from jax import lax
from jax.experimental import pallas as pl
from jax.experimental.pallas import tpu as pltpu
from jax.experimental.pallas import tpu_sc as plsc

N_USERS = 5000
N_ITEMS = 5000
NN = N_USERS + N_ITEMS
D = 128
E = 320000
LAYERS = 3

NC = 2
NS = 16
NW = NC * NS
EPW = E // NW
CH = 80
NCHUNK = EPW // CH
WB_OFF = 624
WB_SZ = 640
DEGW = 128

_MESH = plsc.VectorSubcoreMesh(core_axis_name="c", subcore_axis_name="s")


@functools.partial(
    pl.kernel,
    out_type=jax.ShapeDtypeStruct((NC, NN, DEGW), jnp.float32),
    mesh=_MESH,
    scratch_types=[
        pltpu.VMEM_SHARED((NN, DEGW), jnp.float32),
        pltpu.VMEM((CH,), jnp.int32),
        pltpu.VMEM((CH, DEGW), jnp.float32),
    ],
)
def _degree_kernel(dst_hbm, zeros16_hbm, ones16_hbm, dpart_hbm,
                   hist_sh, idx_v, ones_v):
    c = lax.axis_index("c")
    s = lax.axis_index("s")
    w = s * NC + c
    pltpu.sync_copy(zeros16_hbm.at[pl.ds(s * WB_OFF, WB_SZ)],
                    hist_sh.at[pl.ds(s * WB_OFF, WB_SZ)])
    pltpu.sync_copy(ones16_hbm, ones_v)
    plsc.subcore_barrier()
    base = w * EPW

    def body(i, carry):
        off = pl.multiple_of(base + i * CH, CH)
        pltpu.sync_copy(dst_hbm.at[pl.ds(off, CH)], idx_v)
        pltpu.sync_copy(ones_v, hist_sh.at[idx_v], add=True)
        return carry

    lax.fori_loop(0, NCHUNK, body, 0)
    plsc.subcore_barrier()
    pltpu.sync_copy(hist_sh.at[pl.ds(s * WB_OFF, WB_SZ)],
                    dpart_hbm.at[c, pl.ds(s * WB_OFF, WB_SZ)])


@functools.partial(
    pl.kernel,
    out_type=jax.ShapeDtypeStruct((NC, NN, D), jnp.float32),
    mesh=_MESH,
    scratch_types=[
        pltpu.VMEM_SHARED((NN, D), jnp.float32),
        pltpu.VMEM((CH,), jnp.int32),
        pltpu.VMEM((CH,), jnp.int32),
        pltpu.VMEM((CH, D), jnp.float32),
        pltpu.SemaphoreType.DMA,
    ],
)
def _layer_kernel(x_hbm, src_hbm, dst_hbm, zeros_hbm, ypart_hbm,
                  acc_sh, sidx_v, didx_v, rows_v, sem):
    c = lax.axis_index("c")
    s = lax.axis_index("s")
    w = s * NC + c
    pltpu.sync_copy(zeros_hbm.at[pl.ds(s * WB_OFF, WB_SZ)],
                    acc_sh.at[pl.ds(s * WB_OFF, WB_SZ)])
    plsc.subcore_barrier()
    base = w * EPW

    def body(i, carry):
        off = pl.multiple_of(base + i * CH, CH)
        pltpu.sync_copy(src_hbm.at[pl.ds(off, CH)], sidx_v)
        pltpu.sync_copy(dst_hbm.at[pl.ds(off, CH)], didx_v)
        pltpu.async_copy(x_hbm.at[sidx_v], rows_v, sem).wait()
        pltpu.sync_copy(rows_v, acc_sh.at[didx_v], add=True)
        return carry

    lax.fori_loop(0, NCHUNK, body, 0)
    plsc.subcore_barrier()
    pltpu.sync_copy(acc_sh.at[pl.ds(s * WB_OFF, WB_SZ)],
                    ypart_hbm.at[c, pl.ds(s * WB_OFF, WB_SZ)])


_RB = 1000


def _prep(dpart, emb0):
    def body(dp_ref, emb_ref, s_ref, x_ref):
        d = dp_ref[0, :, 0:1] + dp_ref[1, :, 0:1]
        sv = jnp.where(d > 0.0, lax.rsqrt(jnp.where(d > 0.0, d, 1.0)), 0.0)
        sb = jnp.broadcast_to(sv, (_RB, D))
        s_ref[...] = sb
        x_ref[...] = emb_ref[...] * sb

    return pl.pallas_call(
        body,
        grid=(NN // _RB,),
        in_specs=[
            pl.BlockSpec((NC, _RB, DEGW), lambda i: (0, i, 0)),
            pl.BlockSpec((_RB, D), lambda i: (i, 0)),
        ],
        out_specs=[
            pl.BlockSpec((_RB, D), lambda i: (i, 0)),
            pl.BlockSpec((_RB, D), lambda i: (i, 0)),
        ],
        out_shape=[
            jax.ShapeDtypeStruct((NN, D), jnp.float32),
            jax.ShapeDtypeStruct((NN, D), jnp.float32),
        ],
    )(dpart, emb0)


def _combine(ypart, s_b, acc, last):
    def body(yp_ref, s_ref, a_ref, ao_ref, x_ref):
        y = yp_ref[0] + yp_ref[1]
        e = y * s_ref[...]
        anew = a_ref[...] + e
        if last:
            anew = anew * 0.25
        ao_ref[...] = anew
        x_ref[...] = e * s_ref[...]

    return pl.pallas_call(
        body,
        grid=(NN // _RB,),
        in_specs=[
            pl.BlockSpec((NC, _RB, D), lambda i: (0, i, 0)),
            pl.BlockSpec((_RB, D), lambda i: (i, 0)),
            pl.BlockSpec((_RB, D), lambda i: (i, 0)),
        ],
        out_specs=[
            pl.BlockSpec((_RB, D), lambda i: (i, 0)),
            pl.BlockSpec((_RB, D), lambda i: (i, 0)),
        ],
        out_shape=[
            jax.ShapeDtypeStruct((NN, D), jnp.float32),
            jax.ShapeDtypeStruct((NN, D), jnp.float32),
        ],
    )(ypart, s_b, acc)


def kernel(edge_index, user_emb_weight, item_emb_weight):
    emb0 = jnp.concatenate([user_emb_weight, item_emb_weight], axis=0)
    src = edge_index[0]
    dst = edge_index[1]
    zeros128 = jnp.zeros((NN, D), jnp.float32)
    ones128 = jnp.ones((CH, DEGW), jnp.float32)

    dpart = _degree_kernel(dst, zeros128, ones128)
    s_b, x = _prep(dpart, emb0)
    acc = emb0
    for layer in range(LAYERS):
        ypart = _layer_kernel(x, src, dst, zeros128)
        acc, x = _combine(ypart, s_b, acc, layer == LAYERS - 1)
    return (acc, acc[:N_USERS], acc[N_USERS:])


# speedup vs baseline: 11.7745x; 1.6200x over previous
import functools

import jax
import jax.numpy as jnp
from jax import lax
from jax.experimental import pallas as pl
from jax.experimental.pallas import tpu as pltpu
from jax.experimental.pallas import tpu_sc as plsc

N_USERS = 5000
N_ITEMS = 5000
NN = N_USERS + N_ITEMS
D = 128
E = 320000
LAYERS = 3

NC = 2
NS = 16
NW = NC * NS
EPW = E // NW
CH = 80
NCHUNK = EPW // CH
NPAIR = (NCHUNK - 1) // 2
WB_OFF = 624
WB_SZ = 640
DEGW = 128

_MESH = plsc.VectorSubcoreMesh(core_axis_name="c", subcore_axis_name="s")


@functools.partial(
    pl.kernel,
    out_type=jax.ShapeDtypeStruct((NC, NN, DEGW), jnp.float32),
    mesh=_MESH,
    scratch_types=[
        pltpu.VMEM_SHARED((NN, DEGW), jnp.float32),
        pltpu.VMEM((CH,), jnp.int32),
        pltpu.VMEM((CH,), jnp.int32),
        pltpu.VMEM((CH, DEGW), jnp.float32),
        pltpu.SemaphoreType.DMA,
        pltpu.SemaphoreType.DMA,
        pltpu.SemaphoreType.DMA,
        pltpu.SemaphoreType.DMA,
    ],
)
def _degree_kernel(dst_hbm, zeros_hbm, ones_hbm, dpart_hbm,
                   hist_sh, ia_v, ib_v, ones_v,
                   sem_a, sem_b, sem_ia, sem_ib):
    c = lax.axis_index("c")
    s = lax.axis_index("s")
    w = s * NC + c
    pltpu.sync_copy(zeros_hbm.at[pl.ds(s * WB_OFF, WB_SZ)],
                    hist_sh.at[pl.ds(s * WB_OFF, WB_SZ)])
    pltpu.sync_copy(ones_hbm, ones_v)
    plsc.subcore_barrier()
    base = w * EPW

    def idx_off(i):
        return pl.multiple_of(base + i * CH, 16)

    pltpu.sync_copy(dst_hbm.at[pl.ds(idx_off(0), CH)], ia_v)
    pltpu.async_copy(ones_v, hist_sh.at[ia_v], sem_a, add=True)
    pltpu.async_copy(dst_hbm.at[pl.ds(idx_off(1), CH)], ib_v, sem_ib)

    def body(k, carry):
        i = 2 * k
        pltpu.make_async_copy(dst_hbm.at[pl.ds(idx_off(i + 1), CH)], ib_v,
                              sem_ib).wait()
        pltpu.async_copy(ones_v, hist_sh.at[ib_v], sem_b, add=True)
        pltpu.make_async_copy(ones_v, hist_sh.at[ia_v], sem_a).wait()
        pltpu.async_copy(dst_hbm.at[pl.ds(idx_off(i + 2), CH)], ia_v, sem_ia)
        pltpu.make_async_copy(dst_hbm.at[pl.ds(idx_off(i + 2), CH)], ia_v,
                              sem_ia).wait()
        pltpu.async_copy(ones_v, hist_sh.at[ia_v], sem_a, add=True)
        pltpu.make_async_copy(ones_v, hist_sh.at[ib_v], sem_b).wait()
        pltpu.async_copy(dst_hbm.at[pl.ds(idx_off(i + 3), CH)], ib_v, sem_ib)
        return carry

    lax.fori_loop(0, NPAIR, body, 0)
    pltpu.make_async_copy(ones_v, hist_sh.at[ia_v], sem_a).wait()
    pltpu.make_async_copy(dst_hbm.at[pl.ds(idx_off(NCHUNK), CH)], ib_v,
                          sem_ib).wait()
    plsc.subcore_barrier()
    pltpu.sync_copy(hist_sh.at[pl.ds(s * WB_OFF, WB_SZ)],
                    dpart_hbm.at[c, pl.ds(s * WB_OFF, WB_SZ)])


@functools.partial(
    pl.kernel,
    out_type=jax.ShapeDtypeStruct((NC, NN, D), jnp.float32),
    mesh=_MESH,
    scratch_types=[
        pltpu.VMEM_SHARED((NN, D), jnp.float32),
        pltpu.VMEM((CH,), jnp.int32),
        pltpu.VMEM((CH,), jnp.int32),
        pltpu.VMEM((CH,), jnp.int32),
        pltpu.VMEM((CH,), jnp.int32),
        pltpu.VMEM((CH, D), jnp.float32),
        pltpu.VMEM((CH, D), jnp.float32),
        pltpu.SemaphoreType.DMA,
        pltpu.SemaphoreType.DMA,
        pltpu.SemaphoreType.DMA,
        pltpu.SemaphoreType.DMA,
    ],
)
def _layer_kernel(x_hbm, src_hbm, dst_hbm, zeros_hbm, ypart_hbm,
                  acc_sh, sa_v, da_v, sb_v, db_v, rows_a, rows_b,
                  sem_a, sem_b, sem_ia, sem_ib):
    c = lax.axis_index("c")
    s = lax.axis_index("s")
    w = s * NC + c
    pltpu.sync_copy(zeros_hbm.at[pl.ds(s * WB_OFF, WB_SZ)],
                    acc_sh.at[pl.ds(s * WB_OFF, WB_SZ)])
    plsc.subcore_barrier()
    base = w * EPW

    def idx_off(i):
        return pl.multiple_of(base + i * CH, 16)

    def fetch_idx(i, sref, dref, sem):
        pltpu.async_copy(src_hbm.at[pl.ds(idx_off(i), CH)], sref, sem)
        pltpu.async_copy(dst_hbm.at[pl.ds(idx_off(i), CH)], dref, sem)

    def wait_idx(i, sref, dref, sem):
        pltpu.make_async_copy(src_hbm.at[pl.ds(idx_off(i), CH)], sref,
                              sem).wait()
        pltpu.make_async_copy(dst_hbm.at[pl.ds(idx_off(i), CH)], dref,
                              sem).wait()

    pltpu.sync_copy(src_hbm.at[pl.ds(idx_off(0), CH)], sa_v)
    pltpu.sync_copy(dst_hbm.at[pl.ds(idx_off(0), CH)], da_v)
    pltpu.async_copy(x_hbm.at[sa_v], rows_a, sem_a)
    fetch_idx(1, sb_v, db_v, sem_ib)

    def body(k, carry):
        i = 2 * k
        pltpu.make_async_copy(x_hbm.at[sa_v], rows_a, sem_a).wait()
        wait_idx(i + 1, sb_v, db_v, sem_ib)
        pltpu.async_copy(x_hbm.at[sb_v], rows_b, sem_b)
        pltpu.sync_copy(rows_a, acc_sh.at[da_v], add=True)
        fetch_idx(i + 2, sa_v, da_v, sem_ia)
        pltpu.make_async_copy(x_hbm.at[sb_v], rows_b, sem_b).wait()
        pltpu.sync_copy(rows_b, acc_sh.at[db_v], add=True)
        wait_idx(i + 2, sa_v, da_v, sem_ia)
        pltpu.async_copy(x_hbm.at[sa_v], rows_a, sem_a)
        fetch_idx(i + 3, sb_v, db_v, sem_ib)
        return carry

    lax.fori_loop(0, NPAIR, body, 0)
    pltpu.make_async_copy(x_hbm.at[sa_v], rows_a, sem_a).wait()
    pltpu.sync_copy(rows_a, acc_sh.at[da_v], add=True)
    wait_idx(NCHUNK, sb_v, db_v, sem_ib)
    plsc.subcore_barrier()
    pltpu.sync_copy(acc_sh.at[pl.ds(s * WB_OFF, WB_SZ)],
                    ypart_hbm.at[c, pl.ds(s * WB_OFF, WB_SZ)])


_RB = 1000


def _prep(dpart, emb0):
    def body(dp_ref, emb_ref, s_ref, x_ref):
        d = dp_ref[0, :, 0:1] + dp_ref[1, :, 0:1]
        sv = jnp.where(d > 0.0, lax.rsqrt(jnp.where(d > 0.0, d, 1.0)), 0.0)
        sb = jnp.broadcast_to(sv, (_RB, D))
        s_ref[...] = sb
        x_ref[...] = emb_ref[...] * sb

    return pl.pallas_call(
        body,
        grid=(NN // _RB,),
        in_specs=[
            pl.BlockSpec((NC, _RB, DEGW), lambda i: (0, i, 0)),
            pl.BlockSpec((_RB, D), lambda i: (i, 0)),
        ],
        out_specs=[
            pl.BlockSpec((_RB, D), lambda i: (i, 0)),
            pl.BlockSpec((_RB, D), lambda i: (i, 0)),
        ],
        out_shape=[
            jax.ShapeDtypeStruct((NN, D), jnp.float32),
            jax.ShapeDtypeStruct((NN, D), jnp.float32),
        ],
    )(dpart, emb0)


def _combine(ypart, s_b, acc, last):
    def body(yp_ref, s_ref, a_ref, ao_ref, x_ref):
        y = yp_ref[0] + yp_ref[1]
        e = y * s_ref[...]
        anew = a_ref[...] + e
        if last:
            anew = anew * 0.25
        ao_ref[...] = anew
        x_ref[...] = e * s_ref[...]

    return pl.pallas_call(
        body,
        grid=(NN // _RB,),
        in_specs=[
            pl.BlockSpec((NC, _RB, D), lambda i: (0, i, 0)),
            pl.BlockSpec((_RB, D), lambda i: (i, 0)),
            pl.BlockSpec((_RB, D), lambda i: (i, 0)),
        ],
        out_specs=[
            pl.BlockSpec((_RB, D), lambda i: (i, 0)),
            pl.BlockSpec((_RB, D), lambda i: (i, 0)),
        ],
        out_shape=[
            jax.ShapeDtypeStruct((NN, D), jnp.float32),
            jax.ShapeDtypeStruct((NN, D), jnp.float32),
        ],
    )(ypart, s_b, acc)


def kernel(edge_index, user_emb_weight, item_emb_weight):
    emb0 = jnp.concatenate([user_emb_weight, item_emb_weight], axis=0)
    pad = jnp.zeros((CH,), jnp.int32)
    src = jnp.concatenate([edge_index[0], pad])
    dst = jnp.concatenate([edge_index[1], pad])
    zeros128 = jnp.zeros((NN, D), jnp.float32)
    ones128 = jnp.ones((CH, DEGW), jnp.float32)

    dpart = _degree_kernel(dst, zeros128, ones128)
    s_b, x = _prep(dpart, emb0)
    acc = emb0
    for layer in range(LAYERS):
        ypart = _layer_kernel(x, src, dst, zeros128)
        acc, x = _combine(ypart, s_b, acc, layer == LAYERS - 1)
    return (acc, acc[:N_USERS], acc[N_USERS:])


# speedup vs baseline: 13.1673x; 1.1183x over previous
import functools

import jax
import jax.numpy as jnp
from jax import lax
from jax.experimental import pallas as pl
from jax.experimental.pallas import tpu as pltpu
from jax.experimental.pallas import tpu_sc as plsc

N_USERS = 5000
N_ITEMS = 5000
NN = N_USERS + N_ITEMS
D = 128
E = 320000
LAYERS = 3

NC = 2
NS = 16
NW = NC * NS
EPW = E // NW
CH = 80
NCHUNK = EPW // CH
NPAIR = (NCHUNK - 1) // 2
WB_OFF = 624
WB_SZ = 640
DEGW = 128

_MESH = plsc.VectorSubcoreMesh(core_axis_name="c", subcore_axis_name="s")


@functools.partial(
    pl.kernel,
    out_type=jax.ShapeDtypeStruct((NC, NN, DEGW), jnp.float32),
    mesh=_MESH,
    scratch_types=[
        pltpu.VMEM_SHARED((NN, DEGW), jnp.float32),
        pltpu.VMEM((CH,), jnp.int32),
        pltpu.VMEM((CH,), jnp.int32),
        pltpu.VMEM((CH, DEGW), jnp.float32),
        pltpu.SemaphoreType.DMA,
        pltpu.SemaphoreType.DMA,
        pltpu.SemaphoreType.DMA,
        pltpu.SemaphoreType.DMA,
    ],
)
def _degree_kernel(dst_hbm, zeros_hbm, ones_hbm, dpart_hbm,
                   hist_sh, ia_v, ib_v, ones_v,
                   sem_a, sem_b, sem_ia, sem_ib):
    c = lax.axis_index("c")
    s = lax.axis_index("s")
    w = s * NC + c
    pltpu.sync_copy(zeros_hbm.at[pl.ds(s * WB_OFF, WB_SZ)],
                    hist_sh.at[pl.ds(s * WB_OFF, WB_SZ)])
    pltpu.sync_copy(ones_hbm, ones_v)
    plsc.subcore_barrier()
    base = w * EPW

    def idx_off(i):
        return pl.multiple_of(base + i * CH, 16)

    pltpu.sync_copy(dst_hbm.at[pl.ds(idx_off(0), CH)], ia_v)
    pltpu.async_copy(ones_v, hist_sh.at[ia_v], sem_a, add=True)
    pltpu.async_copy(dst_hbm.at[pl.ds(idx_off(1), CH)], ib_v, sem_ib)

    def body(k, carry):
        i = 2 * k
        pltpu.make_async_copy(dst_hbm.at[pl.ds(idx_off(i + 1), CH)], ib_v,
                              sem_ib).wait()
        pltpu.async_copy(ones_v, hist_sh.at[ib_v], sem_b, add=True)
        pltpu.make_async_copy(ones_v, hist_sh.at[ia_v], sem_a).wait()
        pltpu.async_copy(dst_hbm.at[pl.ds(idx_off(i + 2), CH)], ia_v, sem_ia)
        pltpu.make_async_copy(dst_hbm.at[pl.ds(idx_off(i + 2), CH)], ia_v,
                              sem_ia).wait()
        pltpu.async_copy(ones_v, hist_sh.at[ia_v], sem_a, add=True)
        pltpu.make_async_copy(ones_v, hist_sh.at[ib_v], sem_b).wait()
        pltpu.async_copy(dst_hbm.at[pl.ds(idx_off(i + 3), CH)], ib_v, sem_ib)
        return carry

    lax.fori_loop(0, NPAIR, body, 0)
    pltpu.make_async_copy(ones_v, hist_sh.at[ia_v], sem_a).wait()
    pltpu.make_async_copy(dst_hbm.at[pl.ds(idx_off(NCHUNK), CH)], ib_v,
                          sem_ib).wait()
    plsc.subcore_barrier()
    pltpu.sync_copy(hist_sh.at[pl.ds(s * WB_OFF, WB_SZ)],
                    dpart_hbm.at[c, pl.ds(s * WB_OFF, WB_SZ)])


@functools.partial(
    pl.kernel,
    out_type=jax.ShapeDtypeStruct((NC, NN, D), jnp.float32),
    mesh=_MESH,
    scratch_types=[
        pltpu.VMEM_SHARED((NN, D), jnp.float32),
        pltpu.VMEM((CH,), jnp.int32),
        pltpu.VMEM((CH,), jnp.int32),
        pltpu.VMEM((CH,), jnp.int32),
        pltpu.VMEM((CH,), jnp.int32),
        pltpu.VMEM((CH, D), jnp.float32),
        pltpu.VMEM((CH, D), jnp.float32),
        pltpu.SemaphoreType.DMA,
        pltpu.SemaphoreType.DMA,
        pltpu.SemaphoreType.DMA,
        pltpu.SemaphoreType.DMA,
    ],
)
def _layer_kernel(x_hbm, src_hbm, dst_hbm, zeros_hbm, ypart_hbm,
                  acc_sh, sa_v, da_v, sb_v, db_v, rows_a, rows_b,
                  sem_a, sem_b, sem_ia, sem_ib):
    c = lax.axis_index("c")
    s = lax.axis_index("s")
    w = s * NC + c
    pltpu.sync_copy(zeros_hbm.at[pl.ds(s * WB_OFF, WB_SZ)],
                    acc_sh.at[pl.ds(s * WB_OFF, WB_SZ)])
    plsc.subcore_barrier()
    base = w * EPW

    def idx_off(i):
        return pl.multiple_of(base + i * CH, 16)

    def fetch_idx(i, sref, dref, sem):
        pltpu.async_copy(src_hbm.at[pl.ds(idx_off(i), CH)], sref, sem)
        pltpu.async_copy(dst_hbm.at[pl.ds(idx_off(i), CH)], dref, sem)

    def wait_idx(i, sref, dref, sem):
        pltpu.make_async_copy(src_hbm.at[pl.ds(idx_off(i), CH)], sref,
                              sem).wait()
        pltpu.make_async_copy(dst_hbm.at[pl.ds(idx_off(i), CH)], dref,
                              sem).wait()

    pltpu.sync_copy(src_hbm.at[pl.ds(idx_off(0), CH)], sa_v)
    pltpu.sync_copy(dst_hbm.at[pl.ds(idx_off(0), CH)], da_v)
    pltpu.async_copy(x_hbm.at[sa_v], rows_a, sem_a)
    fetch_idx(1, sb_v, db_v, sem_ib)

    def body(k, carry):
        i = 2 * k
        pltpu.make_async_copy(x_hbm.at[sa_v], rows_a, sem_a).wait()
        wait_idx(i + 1, sb_v, db_v, sem_ib)
        pltpu.async_copy(x_hbm.at[sb_v], rows_b, sem_b)
        pltpu.sync_copy(rows_a, acc_sh.at[da_v], add=True)
        fetch_idx(i + 2, sa_v, da_v, sem_ia)
        pltpu.make_async_copy(x_hbm.at[sb_v], rows_b, sem_b).wait()
        wait_idx(i + 2, sa_v, da_v, sem_ia)
        pltpu.async_copy(x_hbm.at[sa_v], rows_a, sem_a)
        pltpu.sync_copy(rows_b, acc_sh.at[db_v], add=True)
        fetch_idx(i + 3, sb_v, db_v, sem_ib)
        return carry

    lax.fori_loop(0, NPAIR, body, 0)
    pltpu.make_async_copy(x_hbm.at[sa_v], rows_a, sem_a).wait()
    pltpu.sync_copy(rows_a, acc_sh.at[da_v], add=True)
    wait_idx(NCHUNK, sb_v, db_v, sem_ib)
    plsc.subcore_barrier()
    pltpu.sync_copy(acc_sh.at[pl.ds(s * WB_OFF, WB_SZ)],
                    ypart_hbm.at[c, pl.ds(s * WB_OFF, WB_SZ)])


_RB = 1000


def _prep(dpart, emb0):
    def body(dp_ref, emb_ref, s_ref, x_ref):
        d = dp_ref[0, :, 0:1] + dp_ref[1, :, 0:1]
        sv = jnp.where(d > 0.0, lax.rsqrt(jnp.where(d > 0.0, d, 1.0)), 0.0)
        sb = jnp.broadcast_to(sv, (_RB, D))
        s_ref[...] = sb
        x_ref[...] = emb_ref[...] * sb

    return pl.pallas_call(
        body,
        grid=(NN // _RB,),
        in_specs=[
            pl.BlockSpec((NC, _RB, DEGW), lambda i: (0, i, 0)),
            pl.BlockSpec((_RB, D), lambda i: (i, 0)),
        ],
        out_specs=[
            pl.BlockSpec((_RB, D), lambda i: (i, 0)),
            pl.BlockSpec((_RB, D), lambda i: (i, 0)),
        ],
        out_shape=[
            jax.ShapeDtypeStruct((NN, D), jnp.float32),
            jax.ShapeDtypeStruct((NN, D), jnp.float32),
        ],
    )(dpart, emb0)


def _combine(ypart, s_b, acc, last):
    def body(yp_ref, s_ref, a_ref, ao_ref, x_ref):
        y = yp_ref[0] + yp_ref[1]
        e = y * s_ref[...]
        anew = a_ref[...] + e
        if last:
            anew = anew * 0.25
        ao_ref[...] = anew
        x_ref[...] = e * s_ref[...]

    return pl.pallas_call(
        body,
        grid=(NN // _RB,),
        in_specs=[
            pl.BlockSpec((NC, _RB, D), lambda i: (0, i, 0)),
            pl.BlockSpec((_RB, D), lambda i: (i, 0)),
            pl.BlockSpec((_RB, D), lambda i: (i, 0)),
        ],
        out_specs=[
            pl.BlockSpec((_RB, D), lambda i: (i, 0)),
            pl.BlockSpec((_RB, D), lambda i: (i, 0)),
        ],
        out_shape=[
            jax.ShapeDtypeStruct((NN, D), jnp.float32),
            jax.ShapeDtypeStruct((NN, D), jnp.float32),
        ],
    )(ypart, s_b, acc)


def kernel(edge_index, user_emb_weight, item_emb_weight):
    emb0 = jnp.concatenate([user_emb_weight, item_emb_weight], axis=0)
    pad = jnp.zeros((CH,), jnp.int32)
    src = jnp.concatenate([edge_index[0], pad])
    dst = jnp.concatenate([edge_index[1], pad])
    zeros128 = jnp.zeros((NN, D), jnp.float32)
    ones128 = jnp.ones((CH, DEGW), jnp.float32)

    dpart = _degree_kernel(dst, zeros128, ones128)
    s_b, x = _prep(dpart, emb0)
    acc = emb0
    for layer in range(LAYERS):
        ypart = _layer_kernel(x, src, dst, zeros128)
        acc, x = _combine(ypart, s_b, acc, layer == LAYERS - 1)
    return (acc, acc[:N_USERS], acc[N_USERS:])


# speedup vs baseline: 17.9862x; 1.3660x over previous
import functools

import jax
import jax.numpy as jnp
from jax import lax
from jax.experimental import pallas as pl
from jax.experimental.pallas import tpu as pltpu
from jax.experimental.pallas import tpu_sc as plsc

N_USERS = 5000
N_ITEMS = 5000
NN = N_USERS + N_ITEMS
D = 128
E = 320000
LAYERS = 3

NC = 2
NS = 16
NW = NC * NS
EPW = E // NW
CH = 80
NCHUNK = EPW // CH
NPAIR = (NCHUNK - 1) // 2
WB_OFF = 624
WB_SZ = 640
DEGW = 128

_MESH = plsc.VectorSubcoreMesh(core_axis_name="c", subcore_axis_name="s")


@functools.partial(
    pl.kernel,
    out_type=jax.ShapeDtypeStruct((NC, NN, DEGW), jnp.float32),
    mesh=_MESH,
    scratch_types=[
        pltpu.VMEM_SHARED((NN, DEGW), jnp.float32),
        pltpu.VMEM((CH,), jnp.int32),
        pltpu.VMEM((CH,), jnp.int32),
        pltpu.VMEM((CH, DEGW), jnp.float32),
        pltpu.SemaphoreType.DMA,
        pltpu.SemaphoreType.DMA,
        pltpu.SemaphoreType.DMA,
        pltpu.SemaphoreType.DMA,
    ],
)
def _degree_kernel(dst_hbm, zeros_hbm, ones_hbm, dpart_hbm,
                   hist_sh, ia_v, ib_v, ones_v,
                   sem_a, sem_b, sem_ia, sem_ib):
    c = lax.axis_index("c")
    s = lax.axis_index("s")
    w = s * NC + c
    pltpu.sync_copy(zeros_hbm.at[pl.ds(s * WB_OFF, WB_SZ)],
                    hist_sh.at[pl.ds(s * WB_OFF, WB_SZ)])
    pltpu.sync_copy(ones_hbm, ones_v)
    plsc.subcore_barrier()
    base = w * EPW

    def idx_off(i):
        return pl.multiple_of(base + i * CH, 16)

    pltpu.sync_copy(dst_hbm.at[pl.ds(idx_off(0), CH)], ia_v)
    pltpu.async_copy(ones_v, hist_sh.at[ia_v], sem_a, add=True)
    pltpu.async_copy(dst_hbm.at[pl.ds(idx_off(1), CH)], ib_v, sem_ib)

    def body(k, carry):
        i = 2 * k
        pltpu.make_async_copy(dst_hbm.at[pl.ds(idx_off(i + 1), CH)], ib_v,
                              sem_ib).wait()
        pltpu.async_copy(ones_v, hist_sh.at[ib_v], sem_b, add=True)
        pltpu.make_async_copy(ones_v, hist_sh.at[ia_v], sem_a).wait()
        pltpu.async_copy(dst_hbm.at[pl.ds(idx_off(i + 2), CH)], ia_v, sem_ia)
        pltpu.make_async_copy(dst_hbm.at[pl.ds(idx_off(i + 2), CH)], ia_v,
                              sem_ia).wait()
        pltpu.async_copy(ones_v, hist_sh.at[ia_v], sem_a, add=True)
        pltpu.make_async_copy(ones_v, hist_sh.at[ib_v], sem_b).wait()
        pltpu.async_copy(dst_hbm.at[pl.ds(idx_off(i + 3), CH)], ib_v, sem_ib)
        return carry

    lax.fori_loop(0, NPAIR, body, 0)
    pltpu.make_async_copy(ones_v, hist_sh.at[ia_v], sem_a).wait()
    pltpu.make_async_copy(dst_hbm.at[pl.ds(idx_off(NCHUNK), CH)], ib_v,
                          sem_ib).wait()
    plsc.subcore_barrier()
    pltpu.sync_copy(hist_sh.at[pl.ds(s * WB_OFF, WB_SZ)],
                    dpart_hbm.at[c, pl.ds(s * WB_OFF, WB_SZ)])


@functools.partial(
    pl.kernel,
    out_type=jax.ShapeDtypeStruct((NC, NN, D), jnp.float32),
    mesh=_MESH,
    scratch_types=(
        [pltpu.VMEM_SHARED((NN, D), jnp.float32)]
        + [pltpu.VMEM((CH,), jnp.int32) for _ in range(12)]
        + [pltpu.VMEM((CH, D), jnp.float32) for _ in range(3)]
        + [pltpu.SemaphoreType.DMA for _ in range(12)]
    ),
)
def _layer_kernel(x_hbm, src_hbm, dst_hbm, zeros_hbm, ypart_hbm,
                  acc_sh,
                  s0, s1, s2, s3, s4, s5, d0, d1, d2, d3, d4, d5,
                  r0, r1, r2,
                  g0, g1, g2, t0, t1, t2, i0, i1, i2, i3, i4, i5):
    c = lax.axis_index("c")
    s = lax.axis_index("s")
    w = s * NC + c
    pltpu.sync_copy(zeros_hbm.at[pl.ds(s * WB_OFF, WB_SZ)],
                    acc_sh.at[pl.ds(s * WB_OFF, WB_SZ)])
    plsc.subcore_barrier()
    base = w * EPW
    sv = [s0, s1, s2, s3, s4, s5]
    dv = [d0, d1, d2, d3, d4, d5]
    rv = [r0, r1, r2]
    gs = [g0, g1, g2]
    ts = [t0, t1, t2]
    iv = [i0, i1, i2, i3, i4, i5]

    def off(i):
        return pl.multiple_of(base + i * CH, 16)

    def afi(ci, m):
        pltpu.async_copy(src_hbm.at[pl.ds(off(ci), CH)], sv[m], iv[m])
        pltpu.async_copy(dst_hbm.at[pl.ds(off(ci), CH)], dv[m], iv[m])

    def wfi(ci, m):
        pltpu.make_async_copy(src_hbm.at[pl.ds(off(ci), CH)], sv[m],
                              iv[m]).wait()
        pltpu.make_async_copy(dst_hbm.at[pl.ds(off(ci), CH)], dv[m],
                              iv[m]).wait()

    def gather(ci, m, j):
        pltpu.async_copy(x_hbm.at[sv[m]], rv[j], gs[j])

    def wait_gather(m, j):
        pltpu.make_async_copy(x_hbm.at[sv[m]], rv[j], gs[j]).wait()

    def scatter(m, j):
        pltpu.async_copy(rv[j], acc_sh.at[dv[m]], ts[j], add=True)

    def wait_scatter(m, j):
        pltpu.make_async_copy(rv[j], acc_sh.at[dv[m]], ts[j]).wait()

    pltpu.sync_copy(src_hbm.at[pl.ds(off(0), CH)], s0)
    pltpu.sync_copy(dst_hbm.at[pl.ds(off(0), CH)], d0)
    gather(0, 0, 0)
    afi(1, 1)
    afi(2, 2)
    wfi(1, 1); gather(1, 1, 1); wait_gather(0, 0); scatter(0, 0); afi(3, 3)
    wfi(2, 2); gather(2, 2, 2); wait_gather(1, 1); scatter(1, 1); afi(4, 4)
    wfi(3, 3); wait_scatter(0, 0); gather(3, 3, 0)
    wait_gather(2, 2); scatter(2, 2); afi(5, 5)
    wfi(4, 4); wait_scatter(1, 1); gather(4, 4, 1)
    wait_gather(3, 0); scatter(3, 0); afi(6, 0)

    def body(k, carry):
        cc = 6 * k + 5

        def slot(p, j, m, jprev, mprev):
            wfi(cc + p, m)
            wait_scatter((m + 3) % 6, j)
            gather(cc + p, m, j)
            wait_gather(mprev, jprev)
            scatter(mprev, jprev)
            afi(cc + p + 2, (m + 2) % 6)

        slot(0, 2, 5, 1, 4)
        slot(1, 0, 0, 2, 5)
        slot(2, 1, 1, 0, 0)
        slot(3, 2, 2, 1, 1)
        slot(4, 0, 3, 2, 2)
        slot(5, 1, 4, 0, 3)
        return carry

    lax.fori_loop(0, (NCHUNK - 5) // 6, body, 0)
    wait_gather(4, 1)
    scatter(4, 1)
    wait_scatter(2, 2)
    wait_scatter(3, 0)
    wait_scatter(4, 1)
    wfi(NCHUNK, 5)
    wfi(NCHUNK + 1, 0)
    plsc.subcore_barrier()
    pltpu.sync_copy(acc_sh.at[pl.ds(s * WB_OFF, WB_SZ)],
                    ypart_hbm.at[c, pl.ds(s * WB_OFF, WB_SZ)])


_RB = 1000


def _prep(dpart, emb0):
    def body(dp_ref, emb_ref, s_ref, x_ref):
        d = dp_ref[0, :, 0:1] + dp_ref[1, :, 0:1]
        sv = jnp.where(d > 0.0, lax.rsqrt(jnp.where(d > 0.0, d, 1.0)), 0.0)
        sb = jnp.broadcast_to(sv, (_RB, D))
        s_ref[...] = sb
        x_ref[...] = emb_ref[...] * sb

    return pl.pallas_call(
        body,
        grid=(NN // _RB,),
        in_specs=[
            pl.BlockSpec((NC, _RB, DEGW), lambda i: (0, i, 0)),
            pl.BlockSpec((_RB, D), lambda i: (i, 0)),
        ],
        out_specs=[
            pl.BlockSpec((_RB, D), lambda i: (i, 0)),
            pl.BlockSpec((_RB, D), lambda i: (i, 0)),
        ],
        out_shape=[
            jax.ShapeDtypeStruct((NN, D), jnp.float32),
            jax.ShapeDtypeStruct((NN, D), jnp.float32),
        ],
    )(dpart, emb0)


def _combine(ypart, s_b, acc, last):
    def body(yp_ref, s_ref, a_ref, ao_ref, x_ref):
        y = yp_ref[0] + yp_ref[1]
        e = y * s_ref[...]
        anew = a_ref[...] + e
        if last:
            anew = anew * 0.25
        ao_ref[...] = anew
        x_ref[...] = e * s_ref[...]

    return pl.pallas_call(
        body,
        grid=(NN // _RB,),
        in_specs=[
            pl.BlockSpec((NC, _RB, D), lambda i: (0, i, 0)),
            pl.BlockSpec((_RB, D), lambda i: (i, 0)),
            pl.BlockSpec((_RB, D), lambda i: (i, 0)),
        ],
        out_specs=[
            pl.BlockSpec((_RB, D), lambda i: (i, 0)),
            pl.BlockSpec((_RB, D), lambda i: (i, 0)),
        ],
        out_shape=[
            jax.ShapeDtypeStruct((NN, D), jnp.float32),
            jax.ShapeDtypeStruct((NN, D), jnp.float32),
        ],
    )(ypart, s_b, acc)


def kernel(edge_index, user_emb_weight, item_emb_weight):
    emb0 = jnp.concatenate([user_emb_weight, item_emb_weight], axis=0)
    pad = jnp.zeros((2 * CH,), jnp.int32)
    src = jnp.concatenate([edge_index[0], pad])
    dst = jnp.concatenate([edge_index[1], pad])
    zeros128 = jnp.zeros((NN, D), jnp.float32)
    ones128 = jnp.ones((CH, DEGW), jnp.float32)

    dpart = _degree_kernel(dst, zeros128, ones128)
    s_b, x = _prep(dpart, emb0)
    acc = emb0
    for layer in range(LAYERS):
        ypart = _layer_kernel(x, src, dst, zeros128)
        acc, x = _combine(ypart, s_b, acc, layer == LAYERS - 1)
    return (acc, acc[:N_USERS], acc[N_USERS:])


# speedup vs baseline: 17.9881x; 1.0001x over previous
import functools

import jax
import jax.numpy as jnp
from jax import lax
from jax.experimental import pallas as pl
from jax.experimental.pallas import tpu as pltpu
from jax.experimental.pallas import tpu_sc as plsc

N_USERS = 5000
N_ITEMS = 5000
NN = N_USERS + N_ITEMS
D = 128
E = 320000
LAYERS = 3

NC = 2
NS = 16
NW = NC * NS
EPW = E // NW
CH = 80
NCHUNK = EPW // CH
NPAIR = (NCHUNK - 1) // 2
WB_OFF = 624
WB_SZ = 640
DEGW = 128

_MESH = plsc.VectorSubcoreMesh(core_axis_name="c", subcore_axis_name="s")


@functools.partial(
    pl.kernel,
    out_type=jax.ShapeDtypeStruct((NC, NN, DEGW), jnp.float32),
    mesh=_MESH,
    scratch_types=[
        pltpu.VMEM_SHARED((NN, DEGW), jnp.float32),
        pltpu.VMEM((CH,), jnp.int32),
        pltpu.VMEM((CH,), jnp.int32),
        pltpu.VMEM((CH, DEGW), jnp.float32),
        pltpu.SemaphoreType.DMA,
        pltpu.SemaphoreType.DMA,
        pltpu.SemaphoreType.DMA,
        pltpu.SemaphoreType.DMA,
    ],
)
def _degree_kernel(dst_hbm, zeros_hbm, ones_hbm, dpart_hbm,
                   hist_sh, ia_v, ib_v, ones_v,
                   sem_a, sem_b, sem_ia, sem_ib):
    c = lax.axis_index("c")
    s = lax.axis_index("s")
    w = s * NC + c
    pltpu.sync_copy(zeros_hbm.at[pl.ds(s * WB_OFF, WB_SZ)],
                    hist_sh.at[pl.ds(s * WB_OFF, WB_SZ)])
    pltpu.sync_copy(ones_hbm, ones_v)
    plsc.subcore_barrier()
    base = w * EPW

    def idx_off(i):
        return pl.multiple_of(base + i * CH, 16)

    pltpu.sync_copy(dst_hbm.at[pl.ds(idx_off(0), CH)], ia_v)
    pltpu.async_copy(ones_v, hist_sh.at[ia_v], sem_a, add=True)
    pltpu.async_copy(dst_hbm.at[pl.ds(idx_off(1), CH)], ib_v, sem_ib)

    def body(k, carry):
        i = 2 * k
        pltpu.make_async_copy(dst_hbm.at[pl.ds(idx_off(i + 1), CH)], ib_v,
                              sem_ib).wait()
        pltpu.async_copy(ones_v, hist_sh.at[ib_v], sem_b, add=True)
        pltpu.make_async_copy(ones_v, hist_sh.at[ia_v], sem_a).wait()
        pltpu.async_copy(dst_hbm.at[pl.ds(idx_off(i + 2), CH)], ia_v, sem_ia)
        pltpu.make_async_copy(dst_hbm.at[pl.ds(idx_off(i + 2), CH)], ia_v,
                              sem_ia).wait()
        pltpu.async_copy(ones_v, hist_sh.at[ia_v], sem_a, add=True)
        pltpu.make_async_copy(ones_v, hist_sh.at[ib_v], sem_b).wait()
        pltpu.async_copy(dst_hbm.at[pl.ds(idx_off(i + 3), CH)], ib_v, sem_ib)
        return carry

    lax.fori_loop(0, NPAIR, body, 0)
    pltpu.make_async_copy(ones_v, hist_sh.at[ia_v], sem_a).wait()
    pltpu.make_async_copy(dst_hbm.at[pl.ds(idx_off(NCHUNK), CH)], ib_v,
                          sem_ib).wait()
    plsc.subcore_barrier()
    pltpu.sync_copy(hist_sh.at[pl.ds(s * WB_OFF, WB_SZ)],
                    dpart_hbm.at[c, pl.ds(s * WB_OFF, WB_SZ)])


@functools.partial(
    pl.kernel,
    out_type=jax.ShapeDtypeStruct((NC, NN, D), jnp.float32),
    mesh=_MESH,
    scratch_types=(
        [pltpu.VMEM_SHARED((NN, D), jnp.float32)]
        + [pltpu.VMEM((CH,), jnp.int32) for _ in range(12)]
        + [pltpu.VMEM((CH, D), jnp.float32) for _ in range(3)]
        + [pltpu.SemaphoreType.DMA for _ in range(12)]
    ),
)
def _layer_kernel(x_hbm, src_hbm, dst_hbm, zeros_hbm, ypart_hbm,
                  acc_sh,
                  s0, s1, s2, s3, s4, s5, d0, d1, d2, d3, d4, d5,
                  r0, r1, r2,
                  g0, g1, g2, t0, t1, t2, i0, i1, i2, i3, i4, i5):
    c = lax.axis_index("c")
    s = lax.axis_index("s")
    w = s * NC + c
    pltpu.sync_copy(zeros_hbm.at[pl.ds(s * WB_OFF, WB_SZ)],
                    acc_sh.at[pl.ds(s * WB_OFF, WB_SZ)])
    plsc.subcore_barrier()
    base = w * EPW
    sv = [s0, s1, s2, s3, s4, s5]
    dv = [d0, d1, d2, d3, d4, d5]
    rv = [r0, r1, r2]
    gs = [g0, g1, g2]
    ts = [t0, t1, t2]
    iv = [i0, i1, i2, i3, i4, i5]

    def off(i):
        return pl.multiple_of(base + i * CH, 16)

    def afi(ci, m):
        pltpu.async_copy(src_hbm.at[pl.ds(off(ci), CH)], sv[m], iv[m])
        pltpu.async_copy(dst_hbm.at[pl.ds(off(ci), CH)], dv[m], iv[m])

    def wfi(ci, m):
        pltpu.make_async_copy(src_hbm.at[pl.ds(off(ci), CH)], sv[m],
                              iv[m]).wait()
        pltpu.make_async_copy(dst_hbm.at[pl.ds(off(ci), CH)], dv[m],
                              iv[m]).wait()

    def gather(ci, m, j):
        pltpu.async_copy(x_hbm.at[sv[m]], rv[j], gs[j])

    def wait_gather(m, j):
        pltpu.make_async_copy(x_hbm.at[sv[m]], rv[j], gs[j]).wait()

    def scatter(m, j):
        pltpu.async_copy(rv[j], acc_sh.at[dv[m]], ts[j], add=True)

    def wait_scatter(m, j):
        pltpu.make_async_copy(rv[j], acc_sh.at[dv[m]], ts[j]).wait()

    pltpu.sync_copy(src_hbm.at[pl.ds(off(0), CH)], s0)
    pltpu.sync_copy(dst_hbm.at[pl.ds(off(0), CH)], d0)
    gather(0, 0, 0)
    afi(1, 1)
    afi(2, 2)
    wfi(1, 1); gather(1, 1, 1); wait_gather(0, 0); scatter(0, 0); afi(3, 3)
    wfi(2, 2); gather(2, 2, 2); wait_gather(1, 1); scatter(1, 1); afi(4, 4)
    wfi(3, 3); wait_scatter(0, 0); gather(3, 3, 0)
    wait_gather(2, 2); scatter(2, 2); afi(5, 5)
    wfi(4, 4); wait_scatter(1, 1); gather(4, 4, 1)
    wait_gather(3, 0); scatter(3, 0); afi(6, 0)

    def body(k, carry):
        cc = 6 * k + 5

        def slot(p, j, m, jprev, mprev):
            wfi(cc + p, m)
            wait_scatter((m + 3) % 6, j)
            gather(cc + p, m, j)
            wait_gather(mprev, jprev)
            scatter(mprev, jprev)
            afi(cc + p + 2, (m + 2) % 6)

        slot(0, 2, 5, 1, 4)
        slot(1, 0, 0, 2, 5)
        slot(2, 1, 1, 0, 0)
        slot(3, 2, 2, 1, 1)
        slot(4, 0, 3, 2, 2)
        slot(5, 1, 4, 0, 3)
        return carry

    lax.fori_loop(0, (NCHUNK - 5) // 6, body, 0)
    wait_gather(4, 1)
    scatter(4, 1)
    wait_scatter(2, 2)
    wait_scatter(3, 0)
    wait_scatter(4, 1)
    wfi(NCHUNK, 5)
    wfi(NCHUNK + 1, 0)
    plsc.subcore_barrier()
    pltpu.sync_copy(acc_sh.at[pl.ds(s * WB_OFF, WB_SZ)],
                    ypart_hbm.at[c, pl.ds(s * WB_OFF, WB_SZ)])


_RB = 1000


def _prep(dpart, emb0):
    def body(dp_ref, emb_ref, s_ref, x_ref):
        d = dp_ref[0, :, 0:1] + dp_ref[1, :, 0:1]
        sv = jnp.where(d > 0.0, lax.rsqrt(jnp.where(d > 0.0, d, 1.0)), 0.0)
        s_ref[...] = sv
        x_ref[...] = emb_ref[...] * sv

    return pl.pallas_call(
        body,
        grid=(NN // _RB,),
        in_specs=[
            pl.BlockSpec((NC, _RB, DEGW), lambda i: (0, i, 0)),
            pl.BlockSpec((_RB, D), lambda i: (i, 0)),
        ],
        out_specs=[
            pl.BlockSpec((_RB, 1), lambda i: (i, 0)),
            pl.BlockSpec((_RB, D), lambda i: (i, 0)),
        ],
        out_shape=[
            jax.ShapeDtypeStruct((NN, 1), jnp.float32),
            jax.ShapeDtypeStruct((NN, D), jnp.float32),
        ],
    )(dpart, emb0)


def _combine(ypart, s_col, acc, last):
    def body(yp_ref, s_ref, a_ref, ao_ref, *maybe_x):
        y = yp_ref[0] + yp_ref[1]
        sv = s_ref[...]
        e = y * sv
        anew = a_ref[...] + e
        if last:
            anew = anew * 0.25
        else:
            maybe_x[0][...] = e * sv
        ao_ref[...] = anew

    n_out = 1 if last else 2
    return pl.pallas_call(
        body,
        grid=(NN // _RB,),
        in_specs=[
            pl.BlockSpec((NC, _RB, D), lambda i: (0, i, 0)),
            pl.BlockSpec((_RB, 1), lambda i: (i, 0)),
            pl.BlockSpec((_RB, D), lambda i: (i, 0)),
        ],
        out_specs=[pl.BlockSpec((_RB, D), lambda i: (i, 0))] * n_out,
        out_shape=[jax.ShapeDtypeStruct((NN, D), jnp.float32)] * n_out,
    )(ypart, s_col, acc)


def kernel(edge_index, user_emb_weight, item_emb_weight):
    emb0 = jnp.concatenate([user_emb_weight, item_emb_weight], axis=0)
    pad = jnp.zeros((2 * CH,), jnp.int32)
    src = jnp.concatenate([edge_index[0], pad])
    dst = jnp.concatenate([edge_index[1], pad])
    zeros128 = jnp.zeros((NN, D), jnp.float32)
    ones128 = jnp.ones((CH, DEGW), jnp.float32)

    dpart = _degree_kernel(dst, zeros128, ones128)
    s_b, x = _prep(dpart, emb0)
    acc = emb0
    for layer in range(LAYERS):
        ypart = _layer_kernel(x, src, dst, zeros128)
        out = _combine(ypart, s_b, acc, layer == LAYERS - 1)
        acc = out[0]
        x = out[1] if len(out) > 1 else None
    return (acc, acc[:N_USERS], acc[N_USERS:])


# speedup vs baseline: 18.0089x; 1.0012x over previous
import functools

import jax
import jax.numpy as jnp
from jax import lax
from jax.experimental import pallas as pl
from jax.experimental.pallas import tpu as pltpu
from jax.experimental.pallas import tpu_sc as plsc

N_USERS = 5000
N_ITEMS = 5000
NN = N_USERS + N_ITEMS
D = 128
E = 320000
LAYERS = 3

NC = 2
NS = 16
NW = NC * NS
EPW = E // NW
CH = 80
NCHUNK = EPW // CH
NPAIR = (NCHUNK - 1) // 2
WB_OFF = 624
WB_SZ = 640
DEGW = 128

_MESH = plsc.VectorSubcoreMesh(core_axis_name="c", subcore_axis_name="s")


@functools.partial(
    pl.kernel,
    out_type=jax.ShapeDtypeStruct((NC, NN, DEGW), jnp.float32),
    mesh=_MESH,
    scratch_types=[
        pltpu.VMEM_SHARED((NN, DEGW), jnp.float32),
        pltpu.VMEM((CH,), jnp.int32),
        pltpu.VMEM((CH,), jnp.int32),
        pltpu.VMEM((CH, DEGW), jnp.float32),
        pltpu.SemaphoreType.DMA,
        pltpu.SemaphoreType.DMA,
        pltpu.SemaphoreType.DMA,
        pltpu.SemaphoreType.DMA,
    ],
)
def _degree_kernel(dst_hbm, zeros_hbm, ones_hbm, dpart_hbm,
                   hist_sh, ia_v, ib_v, ones_v,
                   sem_a, sem_b, sem_ia, sem_ib):
    c = lax.axis_index("c")
    s = lax.axis_index("s")
    w = s * NC + c
    pltpu.sync_copy(zeros_hbm.at[pl.ds(s * WB_OFF, WB_SZ)],
                    hist_sh.at[pl.ds(s * WB_OFF, WB_SZ)])
    pltpu.sync_copy(ones_hbm, ones_v)
    plsc.subcore_barrier()
    base = w * EPW

    def idx_off(i):
        return pl.multiple_of(base + i * CH, 16)

    pltpu.sync_copy(dst_hbm.at[pl.ds(idx_off(0), CH)], ia_v)
    pltpu.async_copy(ones_v, hist_sh.at[ia_v], sem_a, add=True)
    pltpu.async_copy(dst_hbm.at[pl.ds(idx_off(1), CH)], ib_v, sem_ib)

    def body(k, carry):
        i = 2 * k
        pltpu.make_async_copy(dst_hbm.at[pl.ds(idx_off(i + 1), CH)], ib_v,
                              sem_ib).wait()
        pltpu.async_copy(ones_v, hist_sh.at[ib_v], sem_b, add=True)
        pltpu.make_async_copy(ones_v, hist_sh.at[ia_v], sem_a).wait()
        pltpu.async_copy(dst_hbm.at[pl.ds(idx_off(i + 2), CH)], ia_v, sem_ia)
        pltpu.make_async_copy(dst_hbm.at[pl.ds(idx_off(i + 2), CH)], ia_v,
                              sem_ia).wait()
        pltpu.async_copy(ones_v, hist_sh.at[ia_v], sem_a, add=True)
        pltpu.make_async_copy(ones_v, hist_sh.at[ib_v], sem_b).wait()
        pltpu.async_copy(dst_hbm.at[pl.ds(idx_off(i + 3), CH)], ib_v, sem_ib)
        return carry

    lax.fori_loop(0, NPAIR, body, 0)
    pltpu.make_async_copy(ones_v, hist_sh.at[ia_v], sem_a).wait()
    pltpu.make_async_copy(dst_hbm.at[pl.ds(idx_off(NCHUNK), CH)], ib_v,
                          sem_ib).wait()
    plsc.subcore_barrier()
    pltpu.sync_copy(hist_sh.at[pl.ds(s * WB_OFF, WB_SZ)],
                    dpart_hbm.at[c, pl.ds(s * WB_OFF, WB_SZ)])


@functools.partial(
    pl.kernel,
    out_type=jax.ShapeDtypeStruct((NC, NN, D), jnp.float32),
    mesh=_MESH,
    scratch_types=(
        [pltpu.VMEM_SHARED((NN, D), jnp.float32)]
        + [pltpu.VMEM((CH,), jnp.int32) for _ in range(12)]
        + [pltpu.VMEM((CH, D), jnp.float32) for _ in range(3)]
        + [pltpu.SemaphoreType.DMA for _ in range(12)]
    ),
)
def _layer_kernel(x_hbm, src_hbm, dst_hbm, zeros_hbm, ypart_hbm,
                  acc_sh,
                  s0, s1, s2, s3, s4, s5, d0, d1, d2, d3, d4, d5,
                  r0, r1, r2,
                  g0, g1, g2, t0, t1, t2, i0, i1, i2, i3, i4, i5):
    c = lax.axis_index("c")
    s = lax.axis_index("s")
    w = s * NC + c
    pltpu.sync_copy(zeros_hbm.at[pl.ds(s * WB_OFF, WB_SZ)],
                    acc_sh.at[pl.ds(s * WB_OFF, WB_SZ)])
    plsc.subcore_barrier()
    base = w * EPW
    sv = [s0, s1, s2, s3, s4, s5]
    dv = [d0, d1, d2, d3, d4, d5]
    rv = [r0, r1, r2]
    gs = [g0, g1, g2]
    ts = [t0, t1, t2]
    iv = [i0, i1, i2, i3, i4, i5]

    def off(i):
        return pl.multiple_of(base + i * CH, 16)

    def afi(ci, m):
        pltpu.async_copy(src_hbm.at[pl.ds(off(ci), CH)], sv[m], iv[m])
        pltpu.async_copy(dst_hbm.at[pl.ds(off(ci), CH)], dv[m], iv[m])

    def wfi(ci, m):
        pltpu.make_async_copy(src_hbm.at[pl.ds(off(ci), CH)], sv[m],
                              iv[m]).wait()
        pltpu.make_async_copy(dst_hbm.at[pl.ds(off(ci), CH)], dv[m],
                              iv[m]).wait()

    def gather(ci, m, j):
        pltpu.async_copy(x_hbm.at[sv[m]], rv[j], gs[j])

    def wait_gather(m, j):
        pltpu.make_async_copy(x_hbm.at[sv[m]], rv[j], gs[j]).wait()

    def scatter(m, j):
        pltpu.async_copy(rv[j], acc_sh.at[dv[m]], ts[j], add=True)

    def wait_scatter(m, j):
        pltpu.make_async_copy(rv[j], acc_sh.at[dv[m]], ts[j]).wait()

    pltpu.sync_copy(src_hbm.at[pl.ds(off(0), CH)], s0)
    pltpu.sync_copy(dst_hbm.at[pl.ds(off(0), CH)], d0)
    gather(0, 0, 0)
    afi(1, 1)
    afi(2, 2)
    wfi(1, 1); gather(1, 1, 1); wait_gather(0, 0); scatter(0, 0); afi(3, 3)
    wfi(2, 2); gather(2, 2, 2); wait_gather(1, 1); scatter(1, 1); afi(4, 4)
    wfi(3, 3); wait_scatter(0, 0); gather(3, 3, 0)
    wait_gather(2, 2); scatter(2, 2); afi(5, 5)
    wfi(4, 4); wait_scatter(1, 1); gather(4, 4, 1)
    wait_gather(3, 0); scatter(3, 0); afi(6, 0); afi(7, 1)

    def body(k, carry):
        cc = 6 * k + 5

        def slot(p, j, m, jprev, mprev):
            wfi(cc + p, m)
            wait_scatter((m + 3) % 6, j)
            gather(cc + p, m, j)
            wait_gather(mprev, jprev)
            scatter(mprev, jprev)
            afi(cc + p + 3, (m + 3) % 6)

        slot(0, 2, 5, 1, 4)
        slot(1, 0, 0, 2, 5)
        slot(2, 1, 1, 0, 0)
        slot(3, 2, 2, 1, 1)
        slot(4, 0, 3, 2, 2)
        slot(5, 1, 4, 0, 3)
        return carry

    lax.fori_loop(0, (NCHUNK - 5) // 6, body, 0)
    wait_gather(4, 1)
    scatter(4, 1)
    wait_scatter(2, 2)
    wait_scatter(3, 0)
    wait_scatter(4, 1)
    wfi(NCHUNK, 5)
    wfi(NCHUNK + 1, 0)
    wfi(NCHUNK + 2, 1)
    plsc.subcore_barrier()
    pltpu.sync_copy(acc_sh.at[pl.ds(s * WB_OFF, WB_SZ)],
                    ypart_hbm.at[c, pl.ds(s * WB_OFF, WB_SZ)])


_RB = 1000


def _prep(dpart, emb0):
    def body(dp_ref, emb_ref, s_ref, x_ref):
        d = dp_ref[0, :, 0:1] + dp_ref[1, :, 0:1]
        sv = jnp.where(d > 0.0, lax.rsqrt(jnp.where(d > 0.0, d, 1.0)), 0.0)
        s_ref[...] = sv
        x_ref[...] = emb_ref[...] * sv

    return pl.pallas_call(
        body,
        grid=(NN // _RB,),
        in_specs=[
            pl.BlockSpec((NC, _RB, DEGW), lambda i: (0, i, 0)),
            pl.BlockSpec((_RB, D), lambda i: (i, 0)),
        ],
        out_specs=[
            pl.BlockSpec((_RB, 1), lambda i: (i, 0)),
            pl.BlockSpec((_RB, D), lambda i: (i, 0)),
        ],
        out_shape=[
            jax.ShapeDtypeStruct((NN, 1), jnp.float32),
            jax.ShapeDtypeStruct((NN, D), jnp.float32),
        ],
    )(dpart, emb0)


def _combine(ypart, s_col, acc, last):
    def body(yp_ref, s_ref, a_ref, ao_ref, *maybe_x):
        y = yp_ref[0] + yp_ref[1]
        sv = s_ref[...]
        e = y * sv
        anew = a_ref[...] + e
        if last:
            anew = anew * 0.25
        else:
            maybe_x[0][...] = e * sv
        ao_ref[...] = anew

    n_out = 1 if last else 2
    return pl.pallas_call(
        body,
        grid=(NN // _RB,),
        in_specs=[
            pl.BlockSpec((NC, _RB, D), lambda i: (0, i, 0)),
            pl.BlockSpec((_RB, 1), lambda i: (i, 0)),
            pl.BlockSpec((_RB, D), lambda i: (i, 0)),
        ],
        out_specs=[pl.BlockSpec((_RB, D), lambda i: (i, 0))] * n_out,
        out_shape=[jax.ShapeDtypeStruct((NN, D), jnp.float32)] * n_out,
    )(ypart, s_col, acc)


def kernel(edge_index, user_emb_weight, item_emb_weight):
    emb0 = jnp.concatenate([user_emb_weight, item_emb_weight], axis=0)
    pad = jnp.zeros((3 * CH,), jnp.int32)
    src = jnp.concatenate([edge_index[0], pad])
    dst = jnp.concatenate([edge_index[1], pad])
    zeros128 = jnp.zeros((NN, D), jnp.float32)
    ones128 = jnp.ones((CH, DEGW), jnp.float32)

    dpart = _degree_kernel(dst, zeros128, ones128)
    s_b, x = _prep(dpart, emb0)
    acc = emb0
    for layer in range(LAYERS):
        ypart = _layer_kernel(x, src, dst, zeros128)
        out = _combine(ypart, s_b, acc, layer == LAYERS - 1)
        acc = out[0]
        x = out[1] if len(out) > 1 else None
    return (acc, acc[:N_USERS], acc[N_USERS:])
